# baseline (speedup 1.0000x reference)
import functools

import jax
import jax.numpy as jnp
from jax import lax
from jax.experimental import pallas as pl
from jax.experimental.pallas import tpu as pltpu
from jax.experimental.pallas import tpu_sc as plsc

_ZERO_RATIO = 0.9
_NC = 2
_NS = 16
_L = 16
_CAP2 = 16


def _sc_body(n, k_rank, rows_per_w, x_hbm, o_hbm, buf_a, buf_b, cand_v, hist_v,
             cand2_v, sin_a, sin_b, sout_a, sout_b):
    chunks = n // _L
    bins = 1024
    hist_stride = bins + 1
    cand_stride = chunks + 1
    cand2_stride = _CAP2 + 1
    low_mask = jnp.int32((1 << 22) - 1)
    top = jnp.int32(-2147483648)
    k = jnp.int32(k_rank)
    lane = lax.iota(jnp.int32, _L)
    lane_hist_base = lane * jnp.int32(hist_stride)
    cand_base = lane * jnp.int32(cand_stride)
    cand2_base = lane * jnp.int32(cand2_stride)
    ones = jnp.ones((_L,), jnp.int32)
    zeros = jnp.zeros((_L,), jnp.int32)

    bufs = (buf_a, buf_b)
    sins = (sin_a, sin_b)
    souts = (sout_a, sout_b)

    wid = lax.axis_index("s") * _NC + lax.axis_index("c")
    base_row = wid * rows_per_w

    def zero_hist():
        @plsc.parallel_loop(0, hist_stride * _L // _L, unroll=8)
        def _zero(c):
            hist_v[pl.ds(c * _L, _L)] = zeros

    def scan_hist(rank, raw_order):
        @plsc.parallel_loop(0, bins // _L, unroll=4,
                            carry=(jnp.int32(0), zeros, zeros))
        def _scan(c, carry):
            total, nle_acc, cb_acc = carry
            if raw_order:
                neg_half = c < jnp.int32(bins // _L // 2)
                rc = jnp.where(neg_half, jnp.int32(bins // _L - 1) - c,
                               c - jnp.int32(bins // _L // 2))
            else:
                rc = c
            m = hist_v[pl.ds(rc * _L, _L)]
            for l in range(1, _L):
                m = m + hist_v[pl.ds(l * hist_stride + rc * _L, _L)]
            if raw_order:
                m = jnp.where(neg_half, lax.rev(m, (0,)), m)
            cum = plsc.cumsum(m) + total
            le = cum <= rank
            nle_acc = nle_acc + jnp.where(le, 1, 0)
            cb_acc = jnp.maximum(cb_acc, jnp.where(le, cum, 0))
            return jnp.max(cum), nle_acc, cb_acc

        _, nle_acc, cb_acc = _scan
        return jnp.sum(nle_acc), jnp.max(cb_acc)

    in_dma = [None] * rows_per_w
    out_dma = [None] * rows_per_w
    in_dma[0] = pltpu.async_copy(x_hbm.at[base_row], bufs[0], sins[0])
    if rows_per_w > 1:
        in_dma[1] = pltpu.async_copy(x_hbm.at[base_row + 1], bufs[1], sins[1])

    for i in range(rows_per_w):
        b = i % 2
        in_v = bufs[b]
        in_dma[i].wait()

        zero_hist()

        @plsc.parallel_loop(0, chunks, unroll=16)
        def _hist(c):
            bb = lax.bitcast_convert_type(in_v[pl.ds(c * _L, _L)], jnp.int32)
            raw = lax.shift_right_logical(bb, 22)
            plsc.addupdate_scatter(hist_v, [raw + lane_hist_base], ones)

        if 1 <= i < rows_per_w - 1:
            out_dma[i - 1].wait()
            in_dma[i + 1] = pltpu.async_copy(
                x_hbm.at[base_row + i + 1], bufs[(i + 1) % 2], sins[(i + 1) % 2])

        b1, count_before = scan_hist(k, raw_order=True)
        k2 = k - count_before

        pos_bin = b1 >= jnp.int32(512)
        lo_key = (b1 - jnp.int32(512)) << 22
        hi_key = lo_key + low_mask
        lo_bb = jnp.where(pos_bin, lo_key, ~(hi_key ^ top))
        hi_bb = jnp.where(pos_bin, hi_key, ~(lo_key ^ top))
        xm = jnp.where(pos_bin, 0, -1)

        @plsc.parallel_loop(0, chunks, unroll=16, carry=cand_base)
        def _compact(c, cursor):
            bb = lax.bitcast_convert_type(in_v[pl.ds(c * _L, _L)], jnp.int32)
            hit = (bb >= lo_bb) & (bb <= hi_bb)
            low = (bb ^ xm) & low_mask
            plsc.store_scatter(cand_v, [cursor], low, mask=hit)
            return cursor + jnp.where(hit, 1, 0)

        m_l = _compact - cand_base
        max_m = jnp.max(m_l)

        zero_hist()

        @plsc.parallel_loop(0, max_m, unroll=4)
        def _hist2(j):
            v = plsc.load_gather(cand_v, [cand_base + j])
            d2 = (v >> 12) & jnp.int32(0x3FF)
            plsc.addupdate_scatter(hist_v, [d2 + lane_hist_base], ones,
                                   mask=j < m_l)

        b2, count_before2 = scan_hist(k2, raw_order=False)
        k3 = k2 - count_before2

        @plsc.parallel_loop(0, max_m, unroll=4, carry=zeros)
        def _compact2(j, cur):
            v = plsc.load_gather(cand_v, [cand_base + j])
            hit = (j < m_l) & (((v >> 12) & jnp.int32(0x3FF)) == b2)
            idx = cand2_base + jnp.minimum(cur, jnp.int32(_CAP2))
            plsc.store_scatter(cand2_v, [idx], v & jnp.int32(0xFFF), mask=hit)
            return cur + jnp.where(hit, 1, 0)

        m2_l = _compact2
        max_m2 = jnp.max(m2_l)

        def _fast(_):
            vs = []
            for j in range(_CAP2):
                v = plsc.load_gather(cand2_v, [cand2_base + j])
                vs.append(jnp.where(j < m2_l, v, jnp.int32(1 << 12)))
            res = jnp.int32(0)
            for bit in range(11, -1, -1):
                t = res | jnp.int32(1 << bit)
                cnt = zeros
                for v in vs:
                    cnt = cnt + jnp.where(v < t, 1, 0)
                res = jnp.where(jnp.sum(cnt) <= k3, t, res)
            return res

        def _slow(_):
            res = jnp.int32(0)
            for bit in range(11, -1, -1):
                t = res | jnp.int32(1 << bit)

                def _count(j, acc):
                    v = plsc.load_gather(cand_v, [cand_base + j])
                    ok = ((j < m_l)
                          & (((v >> 12) & jnp.int32(0x3FF)) == b2)
                          & ((v & jnp.int32(0xFFF)) < t))
                    return acc + jnp.where(ok, 1, 0)

                cnt = jnp.sum(lax.fori_loop(0, max_m, _count, zeros))
                res = jnp.where(cnt <= k3, t, res)
            return res

        res12 = lax.cond(max_m2 <= _CAP2, _fast, _slow, 0)

        theta_key = (((b1 - jnp.int32(bins // 2)) << 22)
                     | (b2 << 12) | res12)
        theta_bits = jnp.where(theta_key >= 0, theta_key, ~(theta_key ^ top))
        theta = lax.bitcast_convert_type(
            jnp.broadcast_to(theta_bits, (_L,)), jnp.float32)

        @plsc.parallel_loop(0, chunks, unroll=16)
        def _mask(c):
            x = in_v[pl.ds(c * _L, _L)]
            in_v[pl.ds(c * _L, _L)] = jnp.where(x > theta, x, 0.0)

        out_dma[i] = pltpu.async_copy(in_v, o_hbm.at[base_row + i], souts[b])

    for i in range(max(0, rows_per_w - 2), rows_per_w):
        out_dma[i].wait()


def kernel(X):
    batch, n = X.shape
    k_rank = int((1.0 - _ZERO_RATIO) * n)
    nw = _NC * _NS
    rows_per_w = batch // nw
    mesh = plsc.VectorSubcoreMesh(core_axis_name="c", subcore_axis_name="s")
    return pl.kernel(
        functools.partial(_sc_body, n, k_rank, rows_per_w),
        out_type=jax.ShapeDtypeStruct((batch, n), jnp.float32),
        mesh=mesh,
        compiler_params=pltpu.CompilerParams(needs_layout_passes=False),
        scratch_types=[
            pltpu.VMEM((n,), jnp.float32),
            pltpu.VMEM((n,), jnp.float32),
            pltpu.VMEM(((n // _L + 1) * _L,), jnp.int32),
            pltpu.VMEM((1025 * _L,), jnp.int32),
            pltpu.VMEM(((_CAP2 + 1) * _L,), jnp.int32),
            pltpu.SemaphoreType.DMA,
            pltpu.SemaphoreType.DMA,
            pltpu.SemaphoreType.DMA,
            pltpu.SemaphoreType.DMA,
        ],
    )(X)


# speedup vs baseline: 1.1516x; 1.1516x over previous
import functools

import jax
import jax.numpy as jnp
from jax import lax
from jax.experimental import pallas as pl
from jax.experimental.pallas import tpu as pltpu
from jax.experimental.pallas import tpu_sc as plsc

_ZERO_RATIO = 0.9
_NC = 2
_NS = 16
_L = 16
_CAP2 = 16


def _sc_body(n, k_rank, rows_per_w, x_hbm, o_hbm, buf_a, buf_b, cand_v, hist_v,
             cand2_v, sin_a, sin_b, sout_a, sout_b):
    chunks = n // _L
    bins = 1024
    hist_stride = bins + 1
    cand_stride = chunks + 1
    cand2_stride = _CAP2 + 1
    low_mask = jnp.int32((1 << 22) - 1)
    top = jnp.int32(-2147483648)
    k = jnp.int32(k_rank)
    lane = lax.iota(jnp.int32, _L)
    lane_hist_base = lane * jnp.int32(hist_stride)
    cand_base = lane * jnp.int32(cand_stride)
    cand2_base = lane * jnp.int32(cand2_stride)
    ones = jnp.ones((_L,), jnp.int32)
    zeros = jnp.zeros((_L,), jnp.int32)

    bufs = (buf_a, buf_b)
    sins = (sin_a, sin_b)
    souts = (sout_a, sout_b)

    wid = lax.axis_index("s") * _NC + lax.axis_index("c")
    base_row = wid * rows_per_w

    def zero_hist():
        @plsc.parallel_loop(0, hist_stride * _L // _L, unroll=8)
        def _zero(c):
            hist_v[pl.ds(c * _L, _L)] = zeros

    def scan_hist(rank, raw_order):
        @plsc.parallel_loop(0, bins // _L, unroll=4,
                            carry=(jnp.int32(0), zeros, zeros))
        def _scan(c, carry):
            total, nle_acc, cb_acc = carry
            if raw_order:
                neg_half = c < jnp.int32(bins // _L // 2)
                rc = jnp.where(neg_half, jnp.int32(bins // _L - 1) - c,
                               c - jnp.int32(bins // _L // 2))
            else:
                rc = c
            m = hist_v[pl.ds(rc * _L, _L)]
            hist_v[pl.ds(rc * _L, _L)] = zeros
            for l in range(1, _L):
                m = m + hist_v[pl.ds(l * hist_stride + rc * _L, _L)]
                hist_v[pl.ds(l * hist_stride + rc * _L, _L)] = zeros
            if raw_order:
                m = jnp.where(neg_half, lax.rev(m, (0,)), m)
            cum = plsc.cumsum(m) + total
            le = cum <= rank
            nle_acc = nle_acc + jnp.where(le, 1, 0)
            cb_acc = jnp.maximum(cb_acc, jnp.where(le, cum, 0))
            return jnp.max(cum), nle_acc, cb_acc

        _, nle_acc, cb_acc = _scan
        return jnp.sum(nle_acc), jnp.max(cb_acc)

    in_dma = [None] * rows_per_w
    out_dma = [None] * rows_per_w
    in_dma[0] = pltpu.async_copy(x_hbm.at[base_row], bufs[0], sins[0])
    if rows_per_w > 1:
        in_dma[1] = pltpu.async_copy(x_hbm.at[base_row + 1], bufs[1], sins[1])

    zero_hist()

    for i in range(rows_per_w):
        b = i % 2
        in_v = bufs[b]
        in_dma[i].wait()

        @plsc.parallel_loop(0, chunks, unroll=8)
        def _hist(c):
            bb = lax.bitcast_convert_type(in_v[pl.ds(c * _L, _L)], jnp.int32)
            raw = lax.shift_right_logical(bb, 22)
            plsc.addupdate_scatter(hist_v, [raw + lane_hist_base], ones)

        if 1 <= i < rows_per_w - 1:
            out_dma[i - 1].wait()
            in_dma[i + 1] = pltpu.async_copy(
                x_hbm.at[base_row + i + 1], bufs[(i + 1) % 2], sins[(i + 1) % 2])

        b1, count_before = scan_hist(k, raw_order=True)
        k2 = k - count_before

        pos_bin = b1 >= jnp.int32(512)
        lo_key = (b1 - jnp.int32(512)) << 22
        hi_key = lo_key + low_mask
        lo_bb = jnp.where(pos_bin, lo_key, ~(hi_key ^ top))
        hi_bb = jnp.where(pos_bin, hi_key, ~(lo_key ^ top))
        xm = jnp.where(pos_bin, 0, -1)

        @plsc.parallel_loop(0, chunks, unroll=8, carry=cand_base)
        def _compact(c, cursor):
            bb = lax.bitcast_convert_type(in_v[pl.ds(c * _L, _L)], jnp.int32)
            hit = (bb >= lo_bb) & (bb <= hi_bb)
            low = (bb ^ xm) & low_mask
            plsc.store_scatter(cand_v, [cursor], low, mask=hit)
            return cursor + jnp.where(hit, 1, 0)

        m_l = _compact - cand_base
        max_m = jnp.max(m_l)

        @plsc.parallel_loop(0, max_m, unroll=4)
        def _hist2(j):
            v = plsc.load_gather(cand_v, [cand_base + j])
            d2 = (v >> 12) & jnp.int32(0x3FF)
            plsc.addupdate_scatter(hist_v, [d2 + lane_hist_base], ones,
                                   mask=j < m_l)

        b2, count_before2 = scan_hist(k2, raw_order=False)
        k3 = k2 - count_before2

        @plsc.parallel_loop(0, max_m, unroll=4, carry=zeros)
        def _compact2(j, cur):
            v = plsc.load_gather(cand_v, [cand_base + j])
            hit = (j < m_l) & (((v >> 12) & jnp.int32(0x3FF)) == b2)
            idx = cand2_base + jnp.minimum(cur, jnp.int32(_CAP2))
            plsc.store_scatter(cand2_v, [idx], v & jnp.int32(0xFFF), mask=hit)
            return cur + jnp.where(hit, 1, 0)

        m2_l = _compact2
        max_m2 = jnp.max(m2_l)

        def _fast(_):
            vs = []
            for j in range(_CAP2):
                v = plsc.load_gather(cand2_v, [cand2_base + j])
                vs.append(jnp.where(j < m2_l, v, jnp.int32(1 << 12)))
            res = jnp.int32(0)
            for bit in range(11, -1, -1):
                t = res | jnp.int32(1 << bit)
                cnt = zeros
                for v in vs:
                    cnt = cnt + jnp.where(v < t, 1, 0)
                res = jnp.where(jnp.sum(cnt) <= k3, t, res)
            return res

        def _slow(_):
            res = jnp.int32(0)
            for bit in range(11, -1, -1):
                t = res | jnp.int32(1 << bit)

                def _count(j, acc):
                    v = plsc.load_gather(cand_v, [cand_base + j])
                    ok = ((j < m_l)
                          & (((v >> 12) & jnp.int32(0x3FF)) == b2)
                          & ((v & jnp.int32(0xFFF)) < t))
                    return acc + jnp.where(ok, 1, 0)

                cnt = jnp.sum(lax.fori_loop(0, max_m, _count, zeros))
                res = jnp.where(cnt <= k3, t, res)
            return res

        res12 = lax.cond(max_m2 <= _CAP2, _fast, _slow, 0)

        theta_key = (((b1 - jnp.int32(bins // 2)) << 22)
                     | (b2 << 12) | res12)
        theta_bits = jnp.where(theta_key >= 0, theta_key, ~(theta_key ^ top))
        theta = lax.bitcast_convert_type(
            jnp.broadcast_to(theta_bits, (_L,)), jnp.float32)

        @plsc.parallel_loop(0, chunks, unroll=8)
        def _mask(c):
            x = in_v[pl.ds(c * _L, _L)]
            in_v[pl.ds(c * _L, _L)] = jnp.where(x > theta, x, 0.0)

        out_dma[i] = pltpu.async_copy(in_v, o_hbm.at[base_row + i], souts[b])

    for i in range(max(0, rows_per_w - 2), rows_per_w):
        out_dma[i].wait()


def kernel(X):
    batch, n = X.shape
    k_rank = int((1.0 - _ZERO_RATIO) * n)
    nw = _NC * _NS
    rows_per_w = batch // nw
    mesh = plsc.VectorSubcoreMesh(core_axis_name="c", subcore_axis_name="s")
    return pl.kernel(
        functools.partial(_sc_body, n, k_rank, rows_per_w),
        out_type=jax.ShapeDtypeStruct((batch, n), jnp.float32),
        mesh=mesh,
        compiler_params=pltpu.CompilerParams(needs_layout_passes=False),
        scratch_types=[
            pltpu.VMEM((n,), jnp.float32),
            pltpu.VMEM((n,), jnp.float32),
            pltpu.VMEM(((n // _L + 1) * _L,), jnp.int32),
            pltpu.VMEM((1025 * _L,), jnp.int32),
            pltpu.VMEM(((_CAP2 + 1) * _L,), jnp.int32),
            pltpu.SemaphoreType.DMA,
            pltpu.SemaphoreType.DMA,
            pltpu.SemaphoreType.DMA,
            pltpu.SemaphoreType.DMA,
        ],
    )(X)
